# baseline (speedup 1.0000x reference)
import functools

import jax
import jax.numpy as jnp
from jax.experimental import pallas as pl
from jax.experimental.pallas import tpu as pltpu

_K0 = 0.11831801
_K1 = 0.14776132

_CROP = 5


def _tile_loss(xm, xc, xp, ym, yc, yp):
    mu1 = _K0 * (xm + xp) + _K1 * xc
    mu2 = _K0 * (ym + yp) + _K1 * yc
    c2x = _K0 * (xm * xm + xp * xp) + _K1 * (xc * xc)
    c2y = _K0 * (ym * ym + yp * yp) + _K1 * (yc * yc)
    cxy = _K0 * (xm * ym + xp * yp) + _K1 * (xc * yc)
    return ((c2x - mu1 * mu1) * (c2y - mu2 * mu2)
            - 2.0 * (cxy - mu1 * mu2))


def _loss_body(x_ref, y_ref, o_ref, *, w_blk, H, W):
    j = pl.program_id(1)

    c2 = 2.0 * (2.0 * _K0 + _K1)

    def body(i, acc):
        w = x_ref[0, 0, pl.ds((4 * i - 1) * 8, 48), :]
        v = y_ref[0, 0, pl.ds((4 * i - 1) * 8, 48), :]
        xm, xc, xp = w[7:39], w[8:40], w[9:41]
        ym, yc, yp = v[7:39], v[8:40], v[9:41]
        mu1 = _K0 * (xm + xp) + _K1 * xc
        mu2 = _K0 * (ym + yp) + _K1 * yc
        c2x = _K0 * (xm * xm + xp * xp) + _K1 * (xc * xc)
        c2y = _K0 * (ym * ym + yp * yp) + _K1 * (yc * yc)
        a = c2x - mu1 * mu1
        b = c2y - mu2 * mu2
        m = mu1 * mu2
        return acc + ((a * b + (m + m)) - c2 * (xc * yc))

    z = jnp.zeros((32, w_blk), jnp.float32)
    accc = jax.lax.fori_loop(1, H // 32 - 1, body, z, unroll=2)

    hx = x_ref[0, 0, 0:40, :]
    hy = y_ref[0, 0, 0:40, :]
    head = _tile_loss(hx[4:31], hx[5:32], hx[6:33],
                      hy[4:31], hy[5:32], hy[6:33])
    tx = x_ref[0, 0, H - 40:H, :]
    ty = y_ref[0, 0, H - 40:H, :]
    tail = _tile_loss(tx[7:34], tx[8:35], tx[9:36],
                      ty[7:34], ty[8:35], ty[9:36])

    col = j * w_blk + jax.lax.broadcasted_iota(jnp.int32, (1, w_blk), 1)
    cm = ((col >= _CROP) & (col < W - _CROP)).astype(jnp.float32)

    t31 = jnp.sum(hx[31:32] * hy[31:32] * cm)
    t32 = jnp.sum(hx[32:33] * hy[32:33] * cm)
    tm33 = jnp.sum(tx[7:8] * ty[7:8] * cm)
    tm32 = jnp.sum(tx[8:9] * ty[8:9] * cm)

    s = (jnp.sum(accc * cm) - 2.0 * _K0 * (t31 - t32 - tm33 + tm32)
         + jnp.sum(head * cm) + jnp.sum(tail * cm))
    o_ref[0, 0, :] = jnp.full((128,), s * (1.0 / 128.0), dtype=jnp.float32)


def kernel(X, Y):
    B, C, H, W = X.shape
    w_blk = 512 if W % 512 == 0 else W
    nj = W // w_blk

    out = pl.pallas_call(
        functools.partial(_loss_body, w_blk=w_blk, H=H, W=W),
        out_shape=jax.ShapeDtypeStruct((B, 1, nj * 128), jnp.float32),
        grid=(B, nj),
        in_specs=[
            pl.BlockSpec((1, 1, H, w_blk), lambda b, j: (b, 0, 0, j)),
            pl.BlockSpec((1, 1, H, w_blk), lambda b, j: (b, 0, 0, j)),
        ],
        out_specs=pl.BlockSpec((1, 1, 128), lambda b, j: (b, 0, j)),
        compiler_params=pltpu.CompilerParams(
            dimension_semantics=("parallel", "parallel"),
        ),
        name="ssim_loss",
    )(X, Y)

    n = jnp.float32(H - 2 * _CROP) * jnp.float32(W - 2 * _CROP)
    return jnp.sum(out) / n


# speedup vs baseline: 1.0355x; 1.0355x over previous
import functools

import jax
import jax.numpy as jnp
from jax.experimental import pallas as pl
from jax.experimental.pallas import tpu as pltpu

_K0 = 0.11831801
_K1 = 0.14776132

_CROP = 5


def _tile_loss(xm, xc, xp, ym, yc, yp):
    mu1 = _K0 * (xm + xp) + _K1 * xc
    mu2 = _K0 * (ym + yp) + _K1 * yc
    c2x = _K0 * (xm * xm + xp * xp) + _K1 * (xc * xc)
    c2y = _K0 * (ym * ym + yp * yp) + _K1 * (yc * yc)
    cxy = _K0 * (xm * ym + xp * yp) + _K1 * (xc * yc)
    return ((c2x - mu1 * mu1) * (c2y - mu2 * mu2)
            - 2.0 * (cxy - mu1 * mu2))


def _loss_body(x_ref, y_ref, o_ref, *, w_blk, H, W):
    j = pl.program_id(1)

    def body(i, carry):
        acc13, acc2 = carry
        w = x_ref[0, 0, pl.ds((4 * i - 1) * 8, 48), :]
        v = y_ref[0, 0, pl.ds((4 * i - 1) * 8, 48), :]
        xm, xc, xp = w[7:39], w[8:40], w[9:41]
        ym, yc, yp = v[7:39], v[8:40], v[9:41]
        mu1 = _K0 * (xm + xp) + _K1 * xc
        mu2 = _K0 * (ym + yp) + _K1 * yc
        c2x = _K0 * (xm * xm + xp * xp) + _K1 * (xc * xc)
        c2y = _K0 * (ym * ym + yp * yp) + _K1 * (yc * yc)
        a = c2x - mu1 * mu1
        b = c2y - mu2 * mu2
        m = mu1 * mu2
        acc13 = acc13 + (a * b + (m + m))
        acc2 = acc2 + xc * yc
        return acc13, acc2

    z = jnp.zeros((32, w_blk), jnp.float32)
    acc13, acc2 = jax.lax.fori_loop(1, H // 32 - 1, body, (z, z),
                                    unroll=4)

    hx = x_ref[0, 0, 0:40, :]
    hy = y_ref[0, 0, 0:40, :]
    head = _tile_loss(hx[4:31], hx[5:32], hx[6:33],
                      hy[4:31], hy[5:32], hy[6:33])
    tx = x_ref[0, 0, H - 40:H, :]
    ty = y_ref[0, 0, H - 40:H, :]
    tail = _tile_loss(tx[7:34], tx[8:35], tx[9:36],
                      ty[7:34], ty[8:35], ty[9:36])

    col = j * w_blk + jax.lax.broadcasted_iota(jnp.int32, (1, w_blk), 1)
    cm = ((col >= _CROP) & (col < W - _CROP)).astype(jnp.float32)

    t31 = jnp.sum(hx[31:32] * hy[31:32] * cm)
    t32 = jnp.sum(hx[32:33] * hy[32:33] * cm)
    tm33 = jnp.sum(tx[7:8] * ty[7:8] * cm)
    tm32 = jnp.sum(tx[8:9] * ty[8:9] * cm)
    T = jnp.sum(acc2 * cm)
    sum_cxy = (2.0 * _K0 + _K1) * T + _K0 * (t31 - t32 - tm33 + tm32)

    s = (jnp.sum(acc13 * cm) - 2.0 * sum_cxy
         + jnp.sum(head * cm) + jnp.sum(tail * cm))
    o_ref[0, 0, :] = jnp.full((128,), s * (1.0 / 128.0), dtype=jnp.float32)


def kernel(X, Y):
    B, C, H, W = X.shape
    w_blk = 512 if W % 512 == 0 else W
    nj = W // w_blk

    out = pl.pallas_call(
        functools.partial(_loss_body, w_blk=w_blk, H=H, W=W),
        out_shape=jax.ShapeDtypeStruct((B, 1, nj * 128), jnp.float32),
        grid=(B, nj),
        in_specs=[
            pl.BlockSpec((1, 1, H, w_blk), lambda b, j: (b, 0, 0, j)),
            pl.BlockSpec((1, 1, H, w_blk), lambda b, j: (b, 0, 0, j)),
        ],
        out_specs=pl.BlockSpec((1, 1, 128), lambda b, j: (b, 0, j)),
        compiler_params=pltpu.CompilerParams(
            dimension_semantics=("parallel", "parallel"),
        ),
        name="ssim_loss",
    )(X, Y)

    n = jnp.float32(H - 2 * _CROP) * jnp.float32(W - 2 * _CROP)
    return jnp.sum(out) / n


# speedup vs baseline: 1.0507x; 1.0147x over previous
import functools

import jax
import jax.numpy as jnp
from jax.experimental import pallas as pl
from jax.experimental.pallas import tpu as pltpu

_K0 = 0.11831801
_K1 = 0.14776132

_CROP = 5


def _tile_loss(xm, xc, xp, ym, yc, yp):
    mu1 = _K0 * (xm + xp) + _K1 * xc
    mu2 = _K0 * (ym + yp) + _K1 * yc
    c2x = _K0 * (xm * xm + xp * xp) + _K1 * (xc * xc)
    c2y = _K0 * (ym * ym + yp * yp) + _K1 * (yc * yc)
    cxy = _K0 * (xm * ym + xp * yp) + _K1 * (xc * yc)
    return ((c2x - mu1 * mu1) * (c2y - mu2 * mu2)
            - 2.0 * (cxy - mu1 * mu2))


def _loss_body(x_ref, y_ref, o_ref, *, w_blk, H, W):
    j = pl.program_id(1)

    def body(i, carry):
        acc13, acc2 = carry
        w = x_ref[0, 0, pl.ds((4 * i - 1) * 8, 48), :]
        v = y_ref[0, 0, pl.ds((4 * i - 1) * 8, 48), :]
        xm, xc, xp = w[7:39], w[8:40], w[9:41]
        ym, yc, yp = v[7:39], v[8:40], v[9:41]
        mu1 = _K0 * (xm + xp) + _K1 * xc
        mu2 = _K0 * (ym + yp) + _K1 * yc
        c2x = _K0 * (xm * xm + xp * xp) + _K1 * (xc * xc)
        c2y = _K0 * (ym * ym + yp * yp) + _K1 * (yc * yc)
        a = c2x - mu1 * mu1
        b = c2y - mu2 * mu2
        m = mu1 * mu2
        acc13 = acc13 + (a * b + (m + m))
        acc2 = acc2 + xc * yc
        return acc13, acc2

    z = jnp.zeros((32, w_blk), jnp.float32)
    acc13, acc2 = jax.lax.fori_loop(1, H // 32 - 1, body, (z, z),
                                    unroll=8)

    hx = x_ref[0, 0, 0:40, :]
    hy = y_ref[0, 0, 0:40, :]
    head = _tile_loss(hx[4:31], hx[5:32], hx[6:33],
                      hy[4:31], hy[5:32], hy[6:33])
    tx = x_ref[0, 0, H - 40:H, :]
    ty = y_ref[0, 0, H - 40:H, :]
    tail = _tile_loss(tx[7:34], tx[8:35], tx[9:36],
                      ty[7:34], ty[8:35], ty[9:36])

    col = j * w_blk + jax.lax.broadcasted_iota(jnp.int32, (1, w_blk), 1)
    cm = ((col >= _CROP) & (col < W - _CROP)).astype(jnp.float32)

    t31 = jnp.sum(hx[31:32] * hy[31:32] * cm)
    t32 = jnp.sum(hx[32:33] * hy[32:33] * cm)
    tm33 = jnp.sum(tx[7:8] * ty[7:8] * cm)
    tm32 = jnp.sum(tx[8:9] * ty[8:9] * cm)
    T = jnp.sum(acc2 * cm)
    sum_cxy = (2.0 * _K0 + _K1) * T + _K0 * (t31 - t32 - tm33 + tm32)

    s = (jnp.sum(acc13 * cm) - 2.0 * sum_cxy
         + jnp.sum(head * cm) + jnp.sum(tail * cm))
    o_ref[0, 0, :] = jnp.full((128,), s * (1.0 / 128.0), dtype=jnp.float32)


def kernel(X, Y):
    B, C, H, W = X.shape
    w_blk = 512 if W % 512 == 0 else W
    nj = W // w_blk

    out = pl.pallas_call(
        functools.partial(_loss_body, w_blk=w_blk, H=H, W=W),
        out_shape=jax.ShapeDtypeStruct((B, 1, nj * 128), jnp.float32),
        grid=(B, nj),
        in_specs=[
            pl.BlockSpec((1, 1, H, w_blk), lambda b, j: (b, 0, 0, j)),
            pl.BlockSpec((1, 1, H, w_blk), lambda b, j: (b, 0, 0, j)),
        ],
        out_specs=pl.BlockSpec((1, 1, 128), lambda b, j: (b, 0, j)),
        compiler_params=pltpu.CompilerParams(
            dimension_semantics=("parallel", "parallel"),
        ),
        name="ssim_loss",
    )(X, Y)

    n = jnp.float32(H - 2 * _CROP) * jnp.float32(W - 2 * _CROP)
    return jnp.sum(out) / n


# speedup vs baseline: 1.0621x; 1.0108x over previous
import functools

import jax
import jax.numpy as jnp
from jax.experimental import pallas as pl
from jax.experimental.pallas import tpu as pltpu

_K0 = 0.11831801
_K1 = 0.14776132

_CROP = 5


def _tile_loss(xm, xc, xp, ym, yc, yp):
    mu1 = _K0 * (xm + xp) + _K1 * xc
    mu2 = _K0 * (ym + yp) + _K1 * yc
    c2x = _K0 * (xm * xm + xp * xp) + _K1 * (xc * xc)
    c2y = _K0 * (ym * ym + yp * yp) + _K1 * (yc * yc)
    cxy = _K0 * (xm * ym + xp * yp) + _K1 * (xc * yc)
    return ((c2x - mu1 * mu1) * (c2y - mu2 * mu2)
            - 2.0 * (cxy - mu1 * mu2))


def _loss_body(x_ref, y_ref, o_ref, *, w_blk, H, W):
    j = pl.program_id(1)

    def body(i, carry):
        acc13, acc2 = carry
        w = x_ref[0, 0, pl.ds((4 * i - 1) * 8, 48), :]
        v = y_ref[0, 0, pl.ds((4 * i - 1) * 8, 48), :]
        xm, xc, xp = w[7:39], w[8:40], w[9:41]
        ym, yc, yp = v[7:39], v[8:40], v[9:41]
        mu1 = _K0 * (xm + xp) + _K1 * xc
        mu2 = _K0 * (ym + yp) + _K1 * yc
        c2x = _K0 * (xm * xm + xp * xp) + _K1 * (xc * xc)
        c2y = _K0 * (ym * ym + yp * yp) + _K1 * (yc * yc)
        a = c2x - mu1 * mu1
        b = c2y - mu2 * mu2
        m = mu1 * mu2
        acc13 = acc13 + (a * b + (m + m))
        acc2 = acc2 + xc * yc
        return acc13, acc2

    z = jnp.zeros((32, w_blk), jnp.float32)
    acc13, acc2 = jax.lax.fori_loop(1, H // 32 - 1, body, (z, z),
                                    unroll=16)

    hx = x_ref[0, 0, 0:40, :]
    hy = y_ref[0, 0, 0:40, :]
    head = _tile_loss(hx[4:31], hx[5:32], hx[6:33],
                      hy[4:31], hy[5:32], hy[6:33])
    tx = x_ref[0, 0, H - 40:H, :]
    ty = y_ref[0, 0, H - 40:H, :]
    tail = _tile_loss(tx[7:34], tx[8:35], tx[9:36],
                      ty[7:34], ty[8:35], ty[9:36])

    col = j * w_blk + jax.lax.broadcasted_iota(jnp.int32, (1, w_blk), 1)
    cm = ((col >= _CROP) & (col < W - _CROP)).astype(jnp.float32)

    t31 = jnp.sum(hx[31:32] * hy[31:32] * cm)
    t32 = jnp.sum(hx[32:33] * hy[32:33] * cm)
    tm33 = jnp.sum(tx[7:8] * ty[7:8] * cm)
    tm32 = jnp.sum(tx[8:9] * ty[8:9] * cm)
    T = jnp.sum(acc2 * cm)
    sum_cxy = (2.0 * _K0 + _K1) * T + _K0 * (t31 - t32 - tm33 + tm32)

    s = (jnp.sum(acc13 * cm) - 2.0 * sum_cxy
         + jnp.sum(head * cm) + jnp.sum(tail * cm))
    o_ref[0, 0, :] = jnp.full((128,), s * (1.0 / 128.0), dtype=jnp.float32)


def kernel(X, Y):
    B, C, H, W = X.shape
    w_blk = 512 if W % 512 == 0 else W
    nj = W // w_blk

    out = pl.pallas_call(
        functools.partial(_loss_body, w_blk=w_blk, H=H, W=W),
        out_shape=jax.ShapeDtypeStruct((B, 1, nj * 128), jnp.float32),
        grid=(B, nj),
        in_specs=[
            pl.BlockSpec((1, 1, H, w_blk), lambda b, j: (b, 0, 0, j)),
            pl.BlockSpec((1, 1, H, w_blk), lambda b, j: (b, 0, 0, j)),
        ],
        out_specs=pl.BlockSpec((1, 1, 128), lambda b, j: (b, 0, j)),
        compiler_params=pltpu.CompilerParams(
            dimension_semantics=("parallel", "parallel"),
        ),
        name="ssim_loss",
    )(X, Y)

    n = jnp.float32(H - 2 * _CROP) * jnp.float32(W - 2 * _CROP)
    return jnp.sum(out) / n


# speedup vs baseline: 1.1277x; 1.0618x over previous
import functools

import jax
import jax.numpy as jnp
from jax.experimental import pallas as pl
from jax.experimental.pallas import tpu as pltpu

_K0 = 0.11831801
_K1 = 0.14776132

_CROP = 5


def _tile_loss(xm, xc, xp, ym, yc, yp):
    mu1 = _K0 * (xm + xp) + _K1 * xc
    mu2 = _K0 * (ym + yp) + _K1 * yc
    c2x = _K0 * (xm * xm + xp * xp) + _K1 * (xc * xc)
    c2y = _K0 * (ym * ym + yp * yp) + _K1 * (yc * yc)
    cxy = _K0 * (xm * ym + xp * yp) + _K1 * (xc * yc)
    return ((c2x - mu1 * mu1) * (c2y - mu2 * mu2)
            - 2.0 * (cxy - mu1 * mu2))


def _loss_body(x_ref, y_ref, o_ref, *, w_blk, H, W):
    j = pl.program_id(1)

    def body(i, carry):
        acc13, acc2 = carry
        w = x_ref[0, 0, pl.ds((4 * i - 1) * 8, 48), :]
        v = y_ref[0, 0, pl.ds((4 * i - 1) * 8, 48), :]
        xm, xc, xp = w[7:39], w[8:40], w[9:41]
        ym, yc, yp = v[7:39], v[8:40], v[9:41]
        mu1 = _K0 * (xm + xp) + _K1 * xc
        mu2 = _K0 * (ym + yp) + _K1 * yc
        c2x = _K0 * (xm * xm + xp * xp) + _K1 * (xc * xc)
        c2y = _K0 * (ym * ym + yp * yp) + _K1 * (yc * yc)
        a = c2x - mu1 * mu1
        b = c2y - mu2 * mu2
        m = mu1 * mu2
        acc13 = acc13 + (a * b + (m + m))
        acc2 = acc2 + xc * yc
        return acc13, acc2

    z = jnp.zeros((32, w_blk), jnp.float32)
    acc13, acc2 = jax.lax.fori_loop(1, H // 32 - 1, body, (z, z),
                                    unroll=63)

    hx = x_ref[0, 0, 0:40, :]
    hy = y_ref[0, 0, 0:40, :]
    head = _tile_loss(hx[4:31], hx[5:32], hx[6:33],
                      hy[4:31], hy[5:32], hy[6:33])
    tx = x_ref[0, 0, H - 40:H, :]
    ty = y_ref[0, 0, H - 40:H, :]
    tail = _tile_loss(tx[7:34], tx[8:35], tx[9:36],
                      ty[7:34], ty[8:35], ty[9:36])

    col = j * w_blk + jax.lax.broadcasted_iota(jnp.int32, (1, w_blk), 1)
    cm = ((col >= _CROP) & (col < W - _CROP)).astype(jnp.float32)

    t31 = jnp.sum(hx[31:32] * hy[31:32] * cm)
    t32 = jnp.sum(hx[32:33] * hy[32:33] * cm)
    tm33 = jnp.sum(tx[7:8] * ty[7:8] * cm)
    tm32 = jnp.sum(tx[8:9] * ty[8:9] * cm)
    T = jnp.sum(acc2 * cm)
    sum_cxy = (2.0 * _K0 + _K1) * T + _K0 * (t31 - t32 - tm33 + tm32)

    s = (jnp.sum(acc13 * cm) - 2.0 * sum_cxy
         + jnp.sum(head * cm) + jnp.sum(tail * cm))
    o_ref[0, 0, :] = jnp.full((128,), s * (1.0 / 128.0), dtype=jnp.float32)


def kernel(X, Y):
    B, C, H, W = X.shape
    w_blk = 512 if W % 512 == 0 else W
    nj = W // w_blk

    out = pl.pallas_call(
        functools.partial(_loss_body, w_blk=w_blk, H=H, W=W),
        out_shape=jax.ShapeDtypeStruct((B, 1, nj * 128), jnp.float32),
        grid=(B, nj),
        in_specs=[
            pl.BlockSpec((1, 1, H, w_blk), lambda b, j: (b, 0, 0, j)),
            pl.BlockSpec((1, 1, H, w_blk), lambda b, j: (b, 0, 0, j)),
        ],
        out_specs=pl.BlockSpec((1, 1, 128), lambda b, j: (b, 0, j)),
        compiler_params=pltpu.CompilerParams(
            dimension_semantics=("parallel", "parallel"),
        ),
        name="ssim_loss",
    )(X, Y)

    n = jnp.float32(H - 2 * _CROP) * jnp.float32(W - 2 * _CROP)
    return jnp.sum(out) / n
